# baseline (speedup 1.0000x reference)
import functools

import jax
import jax.numpy as jnp
from jax import lax
from jax.experimental import pallas as pl
from jax.experimental.pallas import tpu as pltpu
from jax.experimental.pallas import tpu_sc as plsc

B, V, K = 128, 100000, 64
NG = B // 8


def _sc_gather_kernel(scores_hbm, allowed_hbm, vals_hbm, alw, slab, vals, gsem):
    c = lax.axis_index("c")
    s = lax.axis_index("s")
    wid = c * 16 + s

    @pl.when(wid < NG)
    def _():
        row0 = pl.multiple_of(wid * 8, 8)
        pltpu.sync_copy(allowed_hbm.at[pl.ds(row0, 8)], alw)
        lane = lax.iota(jnp.int32, 16)

        def grow(r, carry):
            handles = []
            ids = []
            for q in range(K // 16):
                id16 = alw[r, pl.ds(q * 16, 16)]
                ids.append(id16)
                for j in range(16):
                    idv = id16[j]
                    off = pl.multiple_of((idv >> 7) * 128, 128)
                    src = scores_hbm.at[row0 + r].at[pl.ds(off, 128)]
                    handles.append(
                        pltpu.async_copy(src, slab.at[q * 16 + j], gsem))
            for h in handles:
                h.wait()
            for q in range(K // 16):
                off16 = jnp.bitwise_and(ids[q], 127)
                k16 = lane + (q * 16)
                vals[r, pl.ds(q * 16, 16)] = plsc.load_gather(
                    slab, [k16, off16])
            return carry

        lax.fori_loop(0, 8, grow, 0)
        pltpu.sync_copy(vals, vals_hbm.at[pl.ds(row0, 8)])


@jax.jit
def _gather_vals(scores, allowed_ids):
    mesh = plsc.VectorSubcoreMesh(core_axis_name="c", subcore_axis_name="s")
    run = functools.partial(
        pl.kernel,
        out_type=jax.ShapeDtypeStruct((B, K), jnp.float32),
        mesh=mesh,
        compiler_params=pltpu.CompilerParams(needs_layout_passes=False),
        scratch_types=[
            pltpu.VMEM((8, K), jnp.int32),
            pltpu.VMEM((K, 128), jnp.float32),
            pltpu.VMEM((8, K), jnp.float32),
            pltpu.SemaphoreType.DMA,
        ],
    )(_sc_gather_kernel)
    return run(scores, allowed_ids)


def _tc_merge_kernel(alw_ref, vals_ref, out_ref):
    out_ref[...] = jnp.full((8, V), -jnp.inf, dtype=jnp.float32)
    lane = lax.broadcasted_iota(jnp.int32, (1, 128), 1)

    def body(k, carry):
        for r in range(8):
            i = alw_ref[r, k]
            t0 = pl.multiple_of((i >> 7) * 128, 128)
            sl = (pl.ds(r, 1), pl.ds(t0, 128))
            cur = out_ref[sl]
            out_ref[sl] = jnp.where(lane == (i & 127), vals_ref[r, k], cur)
        return carry

    lax.fori_loop(0, K, body, 0)


@jax.jit
def _masked_scores(scores, allowed_ids):
    vals = _gather_vals(scores, allowed_ids)
    return pl.pallas_call(
        _tc_merge_kernel,
        out_shape=jax.ShapeDtypeStruct((B, V), jnp.float32),
        grid=(NG,),
        in_specs=[
            pl.BlockSpec((8, K), lambda i: (i, 0), memory_space=pltpu.SMEM),
            pl.BlockSpec((8, K), lambda i: (i, 0), memory_space=pltpu.SMEM),
        ],
        out_specs=pl.BlockSpec((8, V), lambda i: (i, 0)),
    )(allowed_ids, vals)


def kernel(input_ids, scores, allowed_ids):
    del input_ids
    return _masked_scores(scores, allowed_ids)


# speedup vs baseline: 1.1495x; 1.1495x over previous
import functools

import jax
import jax.numpy as jnp
from jax import lax
from jax.experimental import pallas as pl
from jax.experimental.pallas import tpu as pltpu
from jax.experimental.pallas import tpu_sc as plsc

B, V, K = 128, 100000, 64
VPAD = 100096
NT = 49
CW = NT * 128
HALF = 8 * CW
_CHUNKS0 = tuple((j * CW, NT) for j in range(8))
_CHUNKS1 = tuple((HALF + j * CW, NT) for j in range(7)) + ((HALF + 7 * CW, 47),)


def _sc_mask_kernel(scores_hbm, allowed_hbm, out_hbm,
                    bufa, bufb, alw, slab, vals, gsem, fs0, fs1):
    c = lax.axis_index("c")
    s = lax.axis_index("s")
    wid = c * 16 + s
    g = wid % 16
    half = wid // 16
    row0 = pl.multiple_of(g * 8, 8)

    pltpu.sync_copy(allowed_hbm.at[pl.ds(row0, 8)], alw)

    neg = jnp.full((16,), -jnp.inf, dtype=jnp.float32)
    lane = lax.iota(jnp.int32, 16)

    for buf in (bufa, bufb):
        def ftile(t, carry, buf=buf):
            def frow(r, carry2):
                for i in range(128 // 16):
                    buf[t, r, pl.ds(i * 16, 16)] = neg
                return carry2
            return lax.fori_loop(0, 8, frow, carry)
        lax.fori_loop(0, NT, ftile, 0)

    def grow(r, carry):
        handles = []
        for q in range(K // 16):
            id16 = alw[r, pl.ds(q * 16, 16)]
            for j in range(16):
                idv = id16[j]
                off = pl.multiple_of((idv >> 7) * 128, 128)
                src = scores_hbm.at[row0 + r].at[pl.ds(off, 128)]
                handles.append(pltpu.async_copy(src, slab.at[q * 16 + j], gsem))
        for h in handles:
            h.wait()
        for q in range(K // 16):
            id16 = alw[r, pl.ds(q * 16, 16)]
            off16 = jnp.bitwise_and(id16, 127)
            vals[r, pl.ds(q * 16, 16)] = plsc.load_gather(
                slab, [lane + q * 16, off16])
        return carry

    lax.fori_loop(0, 8, grow, 0)

    def patch(buf, c0, ntiles, restore):
        tbase = c0 // 128

        def body(r, carry):
            r16 = jnp.broadcast_to(r, (16,)).astype(jnp.int32)
            for q in range(K // 16):
                id16 = alw[r, pl.ds(q * 16, 16)]
                t16 = (id16 >> 7) - tbase
                l16 = jnp.bitwise_and(id16, 127)
                m = (t16 >= 0) & (t16 < ntiles)
                v16 = neg if restore else vals[r, pl.ds(q * 16, 16)]
                plsc.store_scatter(buf, [t16, r16, l16], v16, mask=m)
            return carry

        lax.fori_loop(0, 8, body, 0)

    for hsel, chunk_list in ((0, _CHUNKS0), (1, _CHUNKS1)):
        @pl.when(half == hsel)
        def _(chunk_list=chunk_list):
            bufs = (bufa, bufb)
            sems = (fs0, fs1)
            pending = [None, None]
            pend_chunk = [None, None]
            for ci, (c0, ntiles) in enumerate(chunk_list):
                slot = ci % 2
                buf = bufs[slot]
                if pending[slot] is not None:
                    pending[slot].wait()
                    pc0, pnt = pend_chunk[slot]
                    patch(buf, pc0, pnt, restore=True)
                patch(buf, c0, ntiles, restore=False)

                def fire(t, carry, buf=buf, c0=c0, sem=sems[slot]):
                    col = pl.multiple_of(c0 + t * 128, 128)
                    dst = out_hbm.at[pl.ds(row0, 8), pl.ds(col, 128)]
                    pltpu.async_copy(buf.at[t], dst, sem)
                    return carry

                lax.fori_loop(0, ntiles, fire, 0)
                c0d = pl.multiple_of(c0 + 0 * wid, 128)
                span = out_hbm.at[pl.ds(row0, 8), pl.ds(c0d, ntiles * 128)]
                src_dummy = scores_hbm.at[pl.ds(row0, 8), pl.ds(c0d, ntiles * 128)]
                pending[slot] = pltpu.make_async_copy(src_dummy, span, sems[slot])
                pend_chunk[slot] = (c0, ntiles)
            for slot in (0, 1):
                if pending[slot] is not None:
                    pending[slot].wait()


@jax.jit
def _masked_scores(scores, allowed_ids):
    mesh = plsc.VectorSubcoreMesh(core_axis_name="c", subcore_axis_name="s")
    run = functools.partial(
        pl.kernel,
        out_type=jax.ShapeDtypeStruct((B, V), jnp.float32),
        mesh=mesh,
        compiler_params=pltpu.CompilerParams(needs_layout_passes=False),
        scratch_types=[
            pltpu.VMEM((NT, 8, 128), jnp.float32),
            pltpu.VMEM((NT, 8, 128), jnp.float32),
            pltpu.VMEM((8, K), jnp.int32),
            pltpu.VMEM((K, 128), jnp.float32),
            pltpu.VMEM((8, K), jnp.float32),
            pltpu.SemaphoreType.DMA,
            pltpu.SemaphoreType.DMA,
            pltpu.SemaphoreType.DMA,
        ],
    )(_sc_mask_kernel)
    return run(scores, allowed_ids)


def kernel(input_ids, scores, allowed_ids):
    del input_ids
    return _masked_scores(scores, allowed_ids)


# speedup vs baseline: 1.1602x; 1.0093x over previous
import functools

import jax
import jax.numpy as jnp
from jax import lax
from jax.experimental import pallas as pl
from jax.experimental.pallas import tpu as pltpu
from jax.experimental.pallas import tpu_sc as plsc

B, V, K = 128, 100000, 64
VPAD = 100096
NT = 49
CW = NT * 128
HALF = 8 * CW
_CHUNKS0 = tuple((j * CW, NT) for j in range(8))
_CHUNKS1 = tuple((HALF + j * CW, NT) for j in range(7)) + ((HALF + 7 * CW, 47),)


def _sc_mask_kernel(scores_hbm, allowed_hbm, out_hbm,
                    bufa, bufb, alw, slab, vals, gsem, fs0, fs1):
    c = lax.axis_index("c")
    s = lax.axis_index("s")
    wid = c * 16 + s
    g = wid % 16
    half = wid // 16
    row0 = pl.multiple_of(g * 8, 8)

    pltpu.sync_copy(allowed_hbm.at[pl.ds(row0, 8)], alw)

    neg = jnp.full((16,), -jnp.inf, dtype=jnp.float32)
    lane = lax.iota(jnp.int32, 16)

    def fill(buf):
        def ftile(t, carry):
            for r in range(8):
                for i in range(128 // 16):
                    buf[t, r, pl.ds(i * 16, 16)] = neg
            return carry
        lax.fori_loop(0, NT, ftile, 0)

    def fire_slabs(r):
        handles = []
        for q in range(K // 16):
            id16 = alw[r, pl.ds(q * 16, 16)]
            for j in range(16):
                idv = id16[j]
                off = pl.multiple_of((idv >> 7) * 128, 128)
                src = scores_hbm.at[row0 + r].at[pl.ds(off, 128)]
                handles.append(pltpu.async_copy(src, slab.at[q * 16 + j], gsem))
        return handles

    def extract(r):
        for q in range(K // 16):
            id16 = alw[r, pl.ds(q * 16, 16)]
            off16 = jnp.bitwise_and(id16, 127)
            vals[r, pl.ds(q * 16, 16)] = plsc.load_gather(
                slab, [lane + q * 16, off16])

    h0 = fire_slabs(0)
    fill(bufa)
    for h in h0:
        h.wait()
    extract(0)

    def grow(r, carry):
        handles = fire_slabs(r)
        for h in handles:
            h.wait()
        extract(r)
        return carry

    lax.fori_loop(1, 8, grow, 0)

    def patch(buf, c0, ntiles, restore):
        tbase = c0 // 128

        def body(r, carry):
            r16 = jnp.broadcast_to(r, (16,)).astype(jnp.int32)
            for q in range(K // 16):
                id16 = alw[r, pl.ds(q * 16, 16)]
                t16 = (id16 >> 7) - tbase
                l16 = jnp.bitwise_and(id16, 127)
                m = (t16 >= 0) & (t16 < ntiles)
                v16 = neg if restore else vals[r, pl.ds(q * 16, 16)]
                plsc.store_scatter(buf, [t16, r16, l16], v16, mask=m)
            return carry

        lax.fori_loop(0, 8, body, 0)

    for hsel, chunk_list in ((0, _CHUNKS0), (1, _CHUNKS1)):
        @pl.when(half == hsel)
        def _(chunk_list=chunk_list):
            bufs = (bufa, bufb)
            sems = (fs0, fs1)
            pending = [None, None]
            pend_chunk = [None, None]
            for ci, (c0, ntiles) in enumerate(chunk_list):
                slot = ci % 2
                buf = bufs[slot]
                if pending[slot] is not None:
                    pending[slot].wait()
                    pc0, pnt = pend_chunk[slot]
                    patch(buf, pc0, pnt, restore=True)
                patch(buf, c0, ntiles, restore=False)

                def fire(t, carry, buf=buf, c0=c0, sem=sems[slot]):
                    col = pl.multiple_of(c0 + t * 128, 128)
                    dst = out_hbm.at[pl.ds(row0, 8), pl.ds(col, 128)]
                    pltpu.async_copy(buf.at[t], dst, sem)
                    return carry

                lax.fori_loop(0, ntiles, fire, 0)
                if ci == 0:
                    fill(bufb)
                c0d = pl.multiple_of(c0 + 0 * wid, 128)
                span = out_hbm.at[pl.ds(row0, 8), pl.ds(c0d, ntiles * 128)]
                src_dummy = scores_hbm.at[pl.ds(row0, 8), pl.ds(c0d, ntiles * 128)]
                pending[slot] = pltpu.make_async_copy(src_dummy, span, sems[slot])
                pend_chunk[slot] = (c0, ntiles)
            for slot in (0, 1):
                if pending[slot] is not None:
                    pending[slot].wait()


@jax.jit
def _masked_scores(scores, allowed_ids):
    mesh = plsc.VectorSubcoreMesh(core_axis_name="c", subcore_axis_name="s")
    run = functools.partial(
        pl.kernel,
        out_type=jax.ShapeDtypeStruct((B, V), jnp.float32),
        mesh=mesh,
        compiler_params=pltpu.CompilerParams(needs_layout_passes=False),
        scratch_types=[
            pltpu.VMEM((NT, 8, 128), jnp.float32),
            pltpu.VMEM((NT, 8, 128), jnp.float32),
            pltpu.VMEM((8, K), jnp.int32),
            pltpu.VMEM((K, 128), jnp.float32),
            pltpu.VMEM((8, K), jnp.float32),
            pltpu.SemaphoreType.DMA,
            pltpu.SemaphoreType.DMA,
            pltpu.SemaphoreType.DMA,
        ],
    )(_sc_mask_kernel)
    return run(scores, allowed_ids)


def kernel(input_ids, scores, allowed_ids):
    del input_ids
    return _masked_scores(scores, allowed_ids)
